# baseline (speedup 1.0000x reference)
import jax
import jax.numpy as jnp
from jax.experimental import pallas as pl

N = 10000


def _layer_norm(x, g, b):
    m = jnp.mean(x, axis=-1, keepdims=True)
    v = jnp.mean((x - m) ** 2, axis=-1, keepdims=True)
    return (x - m) / jnp.sqrt(v + 1e-5) * g + b


def _gcn_conv(x, row, col, w, Wm, bm):
    h = x @ Wm
    out = jnp.zeros((N, Wm.shape[1]), dtype=x.dtype).at[row].add(w[:, None] * h[col])
    return out + bm


def _att_coef(x, row, col, sort_idx, sorted_keys, dlW, dlb):
    EF = row.shape[0]
    normf = x / jnp.clip(jnp.linalg.norm(x, axis=1, keepdims=True), 1e-12)
    sim = jnp.sum(normf[row] * normf[col], axis=1)
    sim = jnp.where(sim < 0.1, 0.0, sim)
    d00 = jnp.sum(jnp.where((row == 0) & (col == 0), sim, 0.0))
    remove = d00 == 1.0
    sim = jnp.where(remove & (row == col), 0.0, sim)
    rowsum = jax.ops.segment_sum(jnp.abs(sim), row, num_segments=N)
    denom = rowsum[row]
    att = jnp.where(denom > 0, sim / jnp.where(denom > 0, denom, 1.0), 0.0)
    qkeys = col * N + row
    pos = jnp.clip(jnp.searchsorted(sorted_keys, qkeys), 0, EF - 1)
    att_sorted = att[sort_idx]
    att_rev = jnp.where(sorted_keys[pos] == qkeys, att_sorted[pos], 0.0)
    character = jnp.stack([att, att_rev], axis=1)
    ds = jax.nn.sigmoid(character @ dlW.T + dlb)
    ds = jnp.where(ds > 0.5, ds, 0.0)
    ds = jnp.where(-ds > -0.49, -ds, 1.0)
    att = att * ds[:, 0]
    att00 = jnp.sum(jnp.where((row == 0) & (col == 0), att, 0.0))
    deg = jax.ops.segment_sum((att != 0).astype(jnp.float32), row, num_segments=N)
    lam = 1.0 / (deg + 1.0)
    tail = jnp.arange(EF) >= (EF - N)
    att = att + jnp.where((att00 == 0.0) & tail, lam[row], 0.0)
    return jnp.where(att != 0, jnp.exp(att), 0.0)


def _noop_pallas(x):
    def body(x_ref, o_ref):
        o_ref[...] = x_ref[...]
    return pl.pallas_call(body, out_shape=jax.ShapeDtypeStruct(x.shape, x.dtype))(x)


def kernel(x, edge_index, W0, b0, W1, b1, g1, be1, g2, be2, dlW, dlb):
    diag = jnp.arange(N, dtype=edge_index.dtype)
    row = jnp.concatenate([edge_index[0], diag])
    col = jnp.concatenate([edge_index[1], diag])
    keys = row * N + col
    sort_idx = jnp.argsort(keys)
    sorted_keys = keys[sort_idx]
    w = _att_coef(x, row, col, sort_idx, sorted_keys, dlW, dlb)
    h = jax.nn.relu(_layer_norm(_gcn_conv(x, row, col, w, W0, b0), g1, be1))
    w = _att_coef(h, row, col, sort_idx, sorted_keys, dlW, dlb)
    h = jax.nn.relu(_layer_norm(_gcn_conv(h, row, col, w, W1, b1), g2, be2))
    w = _att_coef(h, row, col, sort_idx, sorted_keys, dlW, dlb)
    out = _gcn_conv(h, row, col, w, W1, b1)
    return jax.nn.log_softmax(_noop_pallas(out), axis=-1)


# speedup vs baseline: 1.1214x; 1.1214x over previous
import jax
import jax.numpy as jnp
from jax.experimental import pallas as pl

N = 10000
E = 320000
EF = 330000
EP = 330240
ER = 2580
NPAD = 10112
f32 = jnp.float32
i32 = jnp.int32


def _t1_body(f_ref, w_ref, nf_ref, h_ref):
    f = f_ref[...]
    nrm = jnp.sqrt(jnp.sum(f * f, axis=1, keepdims=True))
    nf_ref[...] = f / jnp.clip(nrm, 1e-12)
    h_ref[...] = jnp.dot(f, w_ref[...], preferred_element_type=f32)


def _t1(f, w):
    return pl.pallas_call(
        _t1_body,
        grid=(NPAD // 128,),
        in_specs=[pl.BlockSpec((128, 128), lambda i: (i, 0)),
                  pl.BlockSpec((128, 128), lambda i: (0, 0))],
        out_specs=[pl.BlockSpec((128, 128), lambda i: (i, 0)),
                   pl.BlockSpec((128, 128), lambda i: (i, 0))],
        out_shape=[jax.ShapeDtypeStruct((NPAD, 128), f32),
                   jax.ShapeDtypeStruct((NPAD, 128), f32)],
    )(f, w)


def _att1_body(sim_ref, dr_ref, srv_ref, dc_ref, ab_ref, att2_ref):
    sim = sim_ref[...]
    dr = dr_ref[...]
    srv = srv_ref[...]
    dc = dc_ref[...]
    a = ab_ref[0, 0]
    b = ab_ref[0, 1]
    c0 = ab_ref[0, 2]
    att = jnp.where(dr > 0, sim / jnp.where(dr > 0, dr, 1.0), 0.0)
    arv = jnp.where(dc > 0, srv / jnp.where(dc > 0, dc, 1.0), 0.0)
    z = att * a + arv * b + c0
    att2_ref[...] = jnp.where(z > 0, att, 0.0)


def _att1(sim, dr, srv, dc, ab):
    blk = lambda: pl.BlockSpec((ER, 128), lambda i: (0, 0))
    return pl.pallas_call(
        _att1_body,
        grid=(1,),
        in_specs=[blk(), blk(), blk(), blk(),
                  pl.BlockSpec((1, 8), lambda i: (0, 0))],
        out_specs=blk(),
        out_shape=jax.ShapeDtypeStruct((ER, 128), f32),
    )(sim, dr, srv, dc, ab)


def _att2_body(att2_ref, lam_ref, tf_ref, att3_ref):
    att2 = att2_ref[...]
    att3 = att2 + tf_ref[...] * lam_ref[...]
    att3_ref[...] = jnp.where(att3 != 0.0, jnp.exp(att3), 0.0)


def _att2(att2, lam, tf):
    blk = lambda: pl.BlockSpec((ER, 128), lambda i: (0, 0))
    return pl.pallas_call(
        _att2_body,
        grid=(1,),
        in_specs=[blk(), blk(), blk()],
        out_specs=blk(),
        out_shape=jax.ShapeDtypeStruct((ER, 128), f32),
    )(att2, lam, tf)


def _scale_body(w_ref, hc_ref, o_ref):
    o_ref[...] = w_ref[...] * hc_ref[...]


def _scale(w, hc):
    return pl.pallas_call(
        _scale_body,
        grid=(EP // 2064,),
        in_specs=[pl.BlockSpec((2064, 1), lambda i: (i, 0)),
                  pl.BlockSpec((2064, 128), lambda i: (i, 0))],
        out_specs=pl.BlockSpec((2064, 128), lambda i: (i, 0)),
        out_shape=jax.ShapeDtypeStruct((EP, 128), f32),
    )(w.reshape(EP, 1), hc)


def _t2_body(p_ref, b_ref, g_ref, be_ref, o_ref):
    o = p_ref[...] + b_ref[...]
    m = jnp.mean(o, axis=1, keepdims=True)
    v = jnp.mean((o - m) ** 2, axis=1, keepdims=True)
    o = (o - m) / jnp.sqrt(v + 1e-5) * g_ref[...] + be_ref[...]
    o_ref[...] = jnp.maximum(o, 0.0)


def _t2(p, b, g, be):
    return pl.pallas_call(
        _t2_body,
        grid=(NPAD // 128,),
        in_specs=[pl.BlockSpec((128, 128), lambda i: (i, 0)),
                  pl.BlockSpec((1, 128), lambda i: (0, 0)),
                  pl.BlockSpec((1, 128), lambda i: (0, 0)),
                  pl.BlockSpec((1, 128), lambda i: (0, 0))],
        out_specs=pl.BlockSpec((128, 128), lambda i: (i, 0)),
        out_shape=jax.ShapeDtypeStruct((NPAD, 128), f32),
    )(p, b.reshape(1, 128), g.reshape(1, 128), be.reshape(1, 128))


def _t2f_body(p_ref, b_ref, o_ref):
    o = p_ref[...] + b_ref[...]
    m = jnp.max(o, axis=1, keepdims=True)
    s = o - m
    o_ref[...] = s - jnp.log(jnp.sum(jnp.exp(s), axis=1, keepdims=True))


def _t2f(p, b):
    return pl.pallas_call(
        _t2f_body,
        grid=(NPAD // 128,),
        in_specs=[pl.BlockSpec((128, 128), lambda i: (i, 0)),
                  pl.BlockSpec((1, 128), lambda i: (0, 0))],
        out_specs=pl.BlockSpec((128, 128), lambda i: (i, 0)),
        out_shape=jax.ShapeDtypeStruct((NPAD, 128), f32),
    )(p, b.reshape(1, 128))


def kernel(x, edge_index, W0, b0, W1, b1, g1, be1, g2, be2, dlW, dlb):
    ei = edge_index.astype(i32)
    diag = jnp.arange(N, dtype=i32)
    row = jnp.concatenate([ei[0], diag])
    col = jnp.concatenate([ei[1], diag])
    keys = row * N + col
    sort_idx = jnp.argsort(keys).astype(i32)
    sorted_keys = keys[sort_idx]
    qkeys = col * N + row
    pos = jnp.clip(jnp.searchsorted(sorted_keys, qkeys), 0, EF - 1)
    match = sorted_keys[pos] == qkeys
    rev = jnp.where(match, sort_idx[pos], EF).astype(i32)

    pad_i = jnp.full((EP - EF,), N, i32)
    rowp = jnp.concatenate([row, pad_i])
    colp = jnp.concatenate([col, pad_i])
    revp = jnp.concatenate([rev, jnp.full((EP - EF,), EF, i32)])
    mask00 = (rowp == 0) & (colp == 0)
    diagp = rowp == colp
    tailp = ((jnp.arange(EP) >= E) & (jnp.arange(EP) < EF)).astype(f32)
    k00 = 1.0 + jnp.sum(((ei[0] == 0) & (ei[1] == 0)).astype(f32))
    ab = jnp.concatenate([dlW[0], dlb, jnp.zeros((5,), f32)]).reshape(1, 8)
    xpad = jnp.zeros((NPAD, 128), f32).at[:N].set(x)

    def branch_remove(f):
        r0 = f[0:1]
        nf = r0 / jnp.clip(jnp.linalg.norm(r0, axis=1, keepdims=True), 1e-12)
        v = jnp.sum(nf * nf, axis=1)[0]
        return (k00 * v) == 1.0

    def layer(f, Wm, bm, gg=None, bb=None, final=False):
        nf, h = _t1(f, Wm)
        rm = branch_remove(f)
        sim = jnp.sum(nf[rowp] * nf[colp], axis=1)
        sim = jnp.where(sim < 0.1, 0.0, sim)
        sim = jnp.where(rm & diagp, 0.0, sim)
        rs = jax.ops.segment_sum(jnp.abs(sim), rowp, num_segments=NPAD + 1)
        dr = rs[rowp]
        dc = rs[colp]
        srv = sim[revp]
        att2 = _att1(sim.reshape(ER, 128), dr.reshape(ER, 128),
                     srv.reshape(ER, 128), dc.reshape(ER, 128),
                     ab).reshape(EP)
        att00 = jnp.sum(jnp.where(mask00, att2, 0.0))
        deg = jax.ops.segment_sum((att2 != 0).astype(f32), rowp,
                                  num_segments=NPAD + 1)
        lam = 1.0 / (deg[rowp] + 1.0)
        tf = tailp * (att00 == 0.0).astype(f32)
        w = _att2(att2.reshape(ER, 128), lam.reshape(ER, 128),
                  tf.reshape(ER, 128)).reshape(EP)
        hc = _scale(w, h[colp])
        out = jax.ops.segment_sum(hc, rowp, num_segments=NPAD + 1)[:NPAD]
        if final:
            return _t2f(out, bm)
        return _t2(out, bm, gg, bb)

    f1 = layer(xpad, W0, b0, g1, be1)
    f2 = layer(f1, W1, b1, g2, be2)
    return layer(f2, W1, b1, final=True)[:N]
